# baseline (speedup 1.0000x reference)
import jax
import jax.numpy as jnp
from jax import lax
from jax.experimental import pallas as pl
from jax.experimental.pallas import tpu as pltpu
from jax.experimental.pallas import tpu_sc as plsc

_K = 8192
_C = 32
_P = 2048
_HW = 1024
_N1 = 512
_N2 = 128
_PT = 256
_E0 = _P + _N1 + _N2 + 128
_T0 = _E0 // _PT
_T1 = _P // _PT
_NROW = (_T0 + _T1) * _PT
_NW = 16
_PPW = _P // _NW


def _nd_scores(e, cb):
    dot = lax.dot_general(e, cb, (((1,), (1,)), ((), ())),
                          preferred_element_type=jnp.float32)
    e2 = jnp.sum(e * e, axis=1, keepdims=True)
    cb2 = jnp.sum(cb * cb, axis=1)[None, :]
    return (2.0 * dot - e2) - cb2


def _matcher_soft(e_ref, cb_ref, zidx_ref, zs_ref):
    nd = _nd_scores(e_ref[0], cb_ref[...])
    zidx_ref[...] = jnp.argmax(nd, axis=-1).astype(jnp.int32)
    m = jnp.max(nd, axis=-1, keepdims=True)
    s = jnp.sum(jnp.exp(nd - m), axis=-1)
    zs_ref[...] = 1.0 / s


def _matcher_hard(e_ref, cb_ref, zidx_ref):
    nd = _nd_scores(e_ref[0], cb_ref[...])
    zidx_ref[...] = jnp.argmax(nd, axis=-1).astype(jnp.int32)


def _matcher_call(etab, cb0, cb1):
    zidx_a, zs_a = pl.pallas_call(
        _matcher_soft,
        grid=(_T0,),
        in_specs=[
            pl.BlockSpec((1, _PT, _C), lambda r: (r, 0, 0)),
            pl.BlockSpec((_K, _C), lambda r: (0, 0)),
        ],
        out_specs=[
            pl.BlockSpec((_PT,), lambda r: (r,)),
            pl.BlockSpec((_PT,), lambda r: (r,)),
        ],
        out_shape=[
            jax.ShapeDtypeStruct((_E0,), jnp.int32),
            jax.ShapeDtypeStruct((_E0,), jnp.float32),
        ],
    )(etab.reshape(_T0, _PT, _C), cb0)
    zidx_b = pl.pallas_call(
        _matcher_hard,
        grid=(_T1,),
        in_specs=[
            pl.BlockSpec((1, _PT, _C), lambda r: (r, 0, 0)),
            pl.BlockSpec((_K, _C), lambda r: (0, 0)),
        ],
        out_specs=pl.BlockSpec((_PT,), lambda r: (r,)),
        out_shape=jax.ShapeDtypeStruct((_P,), jnp.int32),
    )(etab[:_P].reshape(_T1, _PT, _C), cb1)
    return zidx_a, zs_a, zidx_b


def _sc_select_gather(zidx_a, zs_a, zidx_b, etab, cb0, cb1,
                      quant_out, zidx0_out,
                      zs0v, j0v, z2v, zs1v, zs2v, j1v, j2v,
                      idx1v, ridxv, e2v, q1v, q2v, e1v, outv, sem):
    wid = lax.axis_index("s")
    base = wid * _PPW
    b = base // _HW
    hw0 = base % _HW
    cps = [
        pltpu.async_copy(zs_a.at[pl.ds(base, _PPW)], zs0v, sem),
        pltpu.async_copy(zidx_a.at[pl.ds(base, _PPW)], j0v, sem),
        pltpu.async_copy(zidx_b.at[pl.ds(base, _PPW)], z2v, sem),
        pltpu.async_copy(zs_a.at[pl.ds(_P, _N1)], zs1v, sem),
        pltpu.async_copy(zs_a.at[pl.ds(_P + _N1, _N2)], zs2v, sem),
        pltpu.async_copy(zidx_a.at[pl.ds(_P, _N1)], j1v, sem),
        pltpu.async_copy(zidx_a.at[pl.ds(_P + _N1, _N2)], j2v, sem),
        pltpu.async_copy(etab.at[pl.ds(base, _PPW)], e2v, sem),
    ]
    for cp in cps:
        cp.wait()

    lane = lax.broadcasted_iota(jnp.int32, (16,), 0)
    for c in range(_PPW // 16):
        sl = pl.ds(c * 16, 16)
        p = base + c * 16 + lane
        hh = lax.shift_right_logical(p, 5) & 31
        ww = p & 31
        pool1 = b * 256 + lax.shift_right_logical(hh, 1) * 16 \
            + lax.shift_right_logical(ww, 1)
        pool2 = b * 64 + lax.shift_right_logical(hh, 2) * 8 \
            + lax.shift_right_logical(ww, 2)
        a1 = plsc.load_gather(zs1v, [pool1])
        a2 = plsc.load_gather(zs2v, [pool2])
        i1 = plsc.load_gather(j1v, [pool1])
        i2 = plsc.load_gather(j2v, [pool2])
        best = zs0v[sl]
        bidx = j0v[sl]
        ridx = p
        c1 = a1 > best
        best = jnp.where(c1, a1, best)
        bidx = jnp.where(c1, i1, bidx)
        ridx = jnp.where(c1, _P + pool1, ridx)
        c2 = a2 > best
        bidx = jnp.where(c2, i2, bidx)
        ridx = jnp.where(c2, _P + _N1 + pool2, ridx)
        idx1v[sl] = bidx
        ridxv[sl] = ridx

    g1 = pltpu.async_copy(cb0.at[idx1v], q1v, sem)
    g2 = pltpu.async_copy(cb1.at[z2v], q2v, sem)
    g3 = pltpu.async_copy(etab.at[ridxv], e1v, sem)
    g1.wait()
    g2.wait()
    g3.wait()

    for r in range(_PPW):
        rvec = jnp.full((16,), r, jnp.int32)
        for ch in range(_C // 16):
            sl = pl.ds(ch * 16, 16)
            ef = (e1v[r, sl] + e2v[r, sl]) * 0.5
            rq = (q1v[r, sl] + q2v[r, sl]) * 0.5
            plsc.store_scatter(outv, [ch * 16 + lane, rvec], ef + (rq - ef))

    pltpu.sync_copy(outv, quant_out.at[b, :, pl.ds(hw0, _PPW)])
    pltpu.sync_copy(idx1v, zidx0_out.at[b, 0, pl.ds(hw0, _PPW)])
    pltpu.sync_copy(z2v, zidx0_out.at[b, 1, pl.ds(hw0, _PPW)])


def _make_sc_call():
    f32 = jnp.float32
    i32 = jnp.int32
    return pl.kernel(
        _sc_select_gather,
        mesh=plsc.VectorSubcoreMesh(core_axis_name="c", subcore_axis_name="s", num_cores=1),
        out_type=[jax.ShapeDtypeStruct((2, _C, _HW), f32),
                  jax.ShapeDtypeStruct((2, 2, _HW), i32)],
        scratch_types=[
            pltpu.VMEM((_PPW,), f32), pltpu.VMEM((_PPW,), i32),
            pltpu.VMEM((_PPW,), i32),
            pltpu.VMEM((_N1,), f32), pltpu.VMEM((_N2,), f32),
            pltpu.VMEM((_N1,), i32), pltpu.VMEM((_N2,), i32),
            pltpu.VMEM((_PPW,), i32), pltpu.VMEM((_PPW,), i32),
            pltpu.VMEM((_PPW, _C), f32), pltpu.VMEM((_PPW, _C), f32),
            pltpu.VMEM((_PPW, _C), f32), pltpu.VMEM((_PPW, _C), f32),
            pltpu.VMEM((_C, _PPW), f32),
            pltpu.SemaphoreType.DMA,
        ],
        compiler_params=pltpu.CompilerParams(use_tc_tiling_on_sc=False,
                                             needs_layout_passes=False,
                                             skip_device_barrier=True,
                                             disable_bounds_checks=True,
                                             disable_semaphore_checks=True),
    )


def kernel(input, codebook0, codebook1, codebook2, codebook3):
    b, c, h, w = input.shape
    x1 = jax.image.resize(input, (b, c, h // 2, w // 2), method='bilinear')
    x2 = jax.image.resize(input, (b, c, h // 4, w // 4), method='bilinear')
    t0 = jnp.transpose(input, (0, 2, 3, 1)).reshape(_P, _C)
    p1 = jnp.transpose(x1, (0, 2, 3, 1)).reshape(_N1, _C)
    p2 = jnp.transpose(x2, (0, 2, 3, 1)).reshape(_N2, _C)
    etab = jnp.concatenate(
        [t0, p1, p2, jnp.zeros((_E0 - _P - _N1 - _N2, _C), jnp.float32)])

    zidx_a, zs_a, zidx_b = _matcher_call(etab, codebook0, codebook1)
    quant_t, zidx_t = _make_sc_call()(zidx_a, zs_a, zidx_b, etab,
                                      codebook0, codebook1)

    return (input,
            zidx_t.reshape(b, 2, h, w),
            quant_t.reshape(b, c, h, w))


# speedup vs baseline: 1.0181x; 1.0181x over previous
import jax
import jax.numpy as jnp
from jax import lax
from jax.experimental import pallas as pl
from jax.experimental.pallas import tpu as pltpu
from jax.experimental.pallas import tpu_sc as plsc

_K = 8192
_C = 32
_P = 2048
_HW = 1024
_N1 = 512
_N2 = 128
_PT = 256
_E0 = _P + _N1 + _N2 + 128
_T0 = _E0 // _PT
_T1 = _P // _PT
_NROW = (_T0 + _T1) * _PT
_NW = 32
_PPW = _P // _NW


def _nd_scores(e, cb):
    dot = lax.dot_general(e, cb, (((1,), (1,)), ((), ())),
                          preferred_element_type=jnp.float32)
    e2 = jnp.sum(e * e, axis=1, keepdims=True)
    cb2 = jnp.sum(cb * cb, axis=1)[None, :]
    return (2.0 * dot - e2) - cb2


def _matcher_soft(e_ref, cb_ref, zidx_ref, zs_ref):
    nd = _nd_scores(e_ref[0], cb_ref[...])
    zidx_ref[...] = jnp.argmax(nd, axis=-1).astype(jnp.int32)
    m = jnp.max(nd, axis=-1, keepdims=True)
    s = jnp.sum(jnp.exp(nd - m), axis=-1)
    zs_ref[...] = 1.0 / s


def _matcher_hard(e_ref, cb_ref, zidx_ref):
    nd = _nd_scores(e_ref[0], cb_ref[...])
    zidx_ref[...] = jnp.argmax(nd, axis=-1).astype(jnp.int32)


def _matcher_call(etab, cb0, cb1):
    zidx_a, zs_a = pl.pallas_call(
        _matcher_soft,
        grid=(_T0,),
        in_specs=[
            pl.BlockSpec((1, _PT, _C), lambda r: (r, 0, 0)),
            pl.BlockSpec((_K, _C), lambda r: (0, 0)),
        ],
        out_specs=[
            pl.BlockSpec((_PT,), lambda r: (r,)),
            pl.BlockSpec((_PT,), lambda r: (r,)),
        ],
        out_shape=[
            jax.ShapeDtypeStruct((_E0,), jnp.int32),
            jax.ShapeDtypeStruct((_E0,), jnp.float32),
        ],
    )(etab.reshape(_T0, _PT, _C), cb0)
    zidx_b = pl.pallas_call(
        _matcher_hard,
        grid=(_T1,),
        in_specs=[
            pl.BlockSpec((1, _PT, _C), lambda r: (r, 0, 0)),
            pl.BlockSpec((_K, _C), lambda r: (0, 0)),
        ],
        out_specs=pl.BlockSpec((_PT,), lambda r: (r,)),
        out_shape=jax.ShapeDtypeStruct((_P,), jnp.int32),
    )(etab[:_P].reshape(_T1, _PT, _C), cb1)
    return zidx_a, zs_a, zidx_b


def _sc_select_gather(zidx_a, zs_a, zidx_b, etab, cb0, cb1,
                      quant_out, zidx0_out,
                      zs0v, j0v, z2v, zs1v, zs2v, j1v, j2v,
                      idx1v, ridxv, e2v, q1v, q2v, e1v, outv, sem):
    wid = lax.axis_index("s") * 2 + lax.axis_index("c")
    base = wid * _PPW
    b = base // _HW
    hw0 = base % _HW
    cps = [
        pltpu.async_copy(zs_a.at[pl.ds(base, _PPW)], zs0v, sem),
        pltpu.async_copy(zidx_a.at[pl.ds(base, _PPW)], j0v, sem),
        pltpu.async_copy(zidx_b.at[pl.ds(base, _PPW)], z2v, sem),
        pltpu.async_copy(zs_a.at[pl.ds(_P, _N1)], zs1v, sem),
        pltpu.async_copy(zs_a.at[pl.ds(_P + _N1, _N2)], zs2v, sem),
        pltpu.async_copy(zidx_a.at[pl.ds(_P, _N1)], j1v, sem),
        pltpu.async_copy(zidx_a.at[pl.ds(_P + _N1, _N2)], j2v, sem),
        pltpu.async_copy(etab.at[pl.ds(base, _PPW)], e2v, sem),
    ]
    for cp in cps:
        cp.wait()

    lane = lax.broadcasted_iota(jnp.int32, (16,), 0)
    for c in range(_PPW // 16):
        sl = pl.ds(c * 16, 16)
        p = base + c * 16 + lane
        hh = lax.shift_right_logical(p, 5) & 31
        ww = p & 31
        pool1 = b * 256 + lax.shift_right_logical(hh, 1) * 16 \
            + lax.shift_right_logical(ww, 1)
        pool2 = b * 64 + lax.shift_right_logical(hh, 2) * 8 \
            + lax.shift_right_logical(ww, 2)
        a1 = plsc.load_gather(zs1v, [pool1])
        a2 = plsc.load_gather(zs2v, [pool2])
        i1 = plsc.load_gather(j1v, [pool1])
        i2 = plsc.load_gather(j2v, [pool2])
        best = zs0v[sl]
        bidx = j0v[sl]
        ridx = p
        c1 = a1 > best
        best = jnp.where(c1, a1, best)
        bidx = jnp.where(c1, i1, bidx)
        ridx = jnp.where(c1, _P + pool1, ridx)
        c2 = a2 > best
        bidx = jnp.where(c2, i2, bidx)
        ridx = jnp.where(c2, _P + _N1 + pool2, ridx)
        idx1v[sl] = bidx
        ridxv[sl] = ridx

    g1 = pltpu.async_copy(cb0.at[idx1v], q1v, sem)
    g2 = pltpu.async_copy(cb1.at[z2v], q2v, sem)
    g3 = pltpu.async_copy(etab.at[ridxv], e1v, sem)
    g1.wait()
    g2.wait()
    g3.wait()

    for r in range(_PPW):
        rvec = jnp.full((16,), r, jnp.int32)
        for ch in range(_C // 16):
            sl = pl.ds(ch * 16, 16)
            ef = (e1v[r, sl] + e2v[r, sl]) * 0.5
            rq = (q1v[r, sl] + q2v[r, sl]) * 0.5
            plsc.store_scatter(outv, [ch * 16 + lane, rvec], ef + (rq - ef))

    pltpu.sync_copy(outv, quant_out.at[b, :, pl.ds(hw0, _PPW)])
    pltpu.sync_copy(idx1v, zidx0_out.at[b, 0, pl.ds(hw0, _PPW)])
    pltpu.sync_copy(z2v, zidx0_out.at[b, 1, pl.ds(hw0, _PPW)])


def _make_sc_call():
    f32 = jnp.float32
    i32 = jnp.int32
    return pl.kernel(
        _sc_select_gather,
        mesh=plsc.VectorSubcoreMesh(core_axis_name="c", subcore_axis_name="s"),
        out_type=[jax.ShapeDtypeStruct((2, _C, _HW), f32),
                  jax.ShapeDtypeStruct((2, 2, _HW), i32)],
        scratch_types=[
            pltpu.VMEM((_PPW,), f32), pltpu.VMEM((_PPW,), i32),
            pltpu.VMEM((_PPW,), i32),
            pltpu.VMEM((_N1,), f32), pltpu.VMEM((_N2,), f32),
            pltpu.VMEM((_N1,), i32), pltpu.VMEM((_N2,), i32),
            pltpu.VMEM((_PPW,), i32), pltpu.VMEM((_PPW,), i32),
            pltpu.VMEM((_PPW, _C), f32), pltpu.VMEM((_PPW, _C), f32),
            pltpu.VMEM((_PPW, _C), f32), pltpu.VMEM((_PPW, _C), f32),
            pltpu.VMEM((_C, _PPW), f32),
            pltpu.SemaphoreType.DMA,
        ],
        compiler_params=pltpu.CompilerParams(use_tc_tiling_on_sc=False,
                                             needs_layout_passes=False),
    )


def kernel(input, codebook0, codebook1, codebook2, codebook3):
    b, c, h, w = input.shape
    x1 = jax.image.resize(input, (b, c, h // 2, w // 2), method='bilinear')
    x2 = jax.image.resize(input, (b, c, h // 4, w // 4), method='bilinear')
    t0 = jnp.transpose(input, (0, 2, 3, 1)).reshape(_P, _C)
    p1 = jnp.transpose(x1, (0, 2, 3, 1)).reshape(_N1, _C)
    p2 = jnp.transpose(x2, (0, 2, 3, 1)).reshape(_N2, _C)
    etab = jnp.concatenate(
        [t0, p1, p2, jnp.zeros((_E0 - _P - _N1 - _N2, _C), jnp.float32)])

    zidx_a, zs_a, zidx_b = _matcher_call(etab, codebook0, codebook1)
    quant_t, zidx_t = _make_sc_call()(zidx_a, zs_a, zidx_b, etab,
                                      codebook0, codebook1)

    return (input,
            zidx_t.reshape(b, 2, h, w),
            quant_t.reshape(b, c, h, w))
